# baseline (speedup 1.0000x reference)
import functools

import jax
import jax.numpy as jnp
from jax import lax
from jax.experimental import pallas as pl
from jax.experimental.pallas import tpu as pltpu
from jax.experimental.pallas import tpu_sc as plsc

N = 10000
E = 320000
D = 128
H = 128
C = 16
G = 64

NC = 2
NS = 16
NPC = N // NC
ACCR = NPC + NS
EPT = E // NS
K = 80
NCH = EPT // K
DW = 128
NB = 2

WB = 312
WBT = NPC - 15 * WB
ZR = 312
ZRT = ACCR - 15 * ZR

_MESH = dict(core_axis_name="c", subcore_axis_name="s")
_F32 = jnp.float32
_HIGH = lax.Precision.HIGHEST


def _remap(dst_v, dst2_v, c, s):
    base = c * NPC
    dump = NPC + s

    def rm(j, carry):
        for l in range(K // 16):
            v = dst_v[j, pl.ds(l * 16, 16)]
            inr = (v >= base) & (v < base + NPC)
            dst2_v[j, pl.ds(l * 16, 16)] = jnp.where(inr, v - base, dump)
        return carry

    lax.fori_loop(0, NCH, rm, 0)


def _zero_init(z_hbm, acc_sh, s):
    zoff = pl.multiple_of(s * ZR, 8)

    @pl.when(s < 15)
    def _():
        pltpu.sync_copy(z_hbm.at[pl.ds(zoff, ZR)], acc_sh.at[pl.ds(zoff, ZR)])

    @pl.when(s == 15)
    def _():
        pltpu.sync_copy(z_hbm.at[pl.ds(15 * ZR, ZRT)],
                        acc_sh.at[pl.ds(15 * ZR, ZRT)])


def _writeback(acc_sh, out_hbm, c, s):
    woff = pl.multiple_of(s * WB, 8)
    base = pl.multiple_of(c * NPC, 8)

    @pl.when(s < 15)
    def _():
        pltpu.sync_copy(acc_sh.at[pl.ds(woff, WB)],
                        out_hbm.at[pl.ds(base + woff, WB)])

    @pl.when(s == 15)
    def _():
        pltpu.sync_copy(acc_sh.at[pl.ds(15 * WB, WBT)],
                        out_hbm.at[pl.ds(base + 15 * WB, WBT)])


def _sc_deg(dstr, zeros16, ones16):
    @functools.partial(
        pl.kernel,
        out_type=jax.ShapeDtypeStruct((N, DW), _F32),
        mesh=plsc.VectorSubcoreMesh(**_MESH),
        scratch_types=[
            pltpu.VMEM((NCH, K), jnp.int32),
            pltpu.VMEM((NCH, K), jnp.int32),
            pltpu.VMEM((K, DW), _F32),
            pltpu.VMEM_SHARED((ACCR, DW), _F32),
            pltpu.SemaphoreType.DMA,
        ],
    )
    def body(dst_hbm, z_hbm, ones_hbm, out_hbm, dst_v, dst2_v, ones_v,
             acc_sh, sem):
        c = lax.axis_index("c")
        s = lax.axis_index("s")
        pltpu.sync_copy(dst_hbm.at[s], dst_v)
        pltpu.sync_copy(ones_hbm, ones_v)
        _remap(dst_v, dst2_v, c, s)
        _zero_init(z_hbm, acc_sh, s)
        plsc.subcore_barrier()

        def it(j, carry):
            pltpu.sync_copy(ones_v, acc_sh.at[dst2_v.at[j]], add=True)
            return carry

        lax.fori_loop(0, NCH, it, 0)
        plsc.subcore_barrier()
        _writeback(acc_sh, out_hbm, c, s)

    return body(dstr, zeros16, ones16)


def _sc_agg(h, srcr, dstr, zeros):
    @functools.partial(
        pl.kernel,
        out_type=jax.ShapeDtypeStruct((N, H), _F32),
        mesh=plsc.VectorSubcoreMesh(**_MESH),
        scratch_types=[
            pltpu.VMEM((NCH, K), jnp.int32),
            pltpu.VMEM((NCH, K), jnp.int32),
            pltpu.VMEM((NB, K, H), _F32),
            pltpu.VMEM_SHARED((ACCR, H), _F32),
            pltpu.SemaphoreType.DMA((NB,)),
            pltpu.SemaphoreType.DMA((NB,)),
        ],
    )
    def body(h_hbm, src_hbm, dst_hbm, z_hbm, out_hbm,
             src_v, dst2_v, rows_v, acc_sh, gsems, ssems):
        c = lax.axis_index("c")
        s = lax.axis_index("s")
        pltpu.sync_copy(dst_hbm.at[s], dst2_v)
        _remap(dst2_v, dst2_v, c, s)
        pltpu.sync_copy(src_hbm.at[s], src_v)
        _zero_init(z_hbm, acc_sh, s)

        pltpu.async_copy(h_hbm.at[src_v.at[0]], rows_v.at[0], gsems.at[0])
        plsc.subcore_barrier()

        def it(j, carry):
            cur = lax.rem(j, NB)
            nxt = lax.rem(j + 1, NB)

            @pl.when(j >= NB - 1)
            def _():
                jo = j - (NB - 1)
                ko = lax.rem(jo, NB)
                pltpu.make_async_copy(rows_v.at[ko],
                                      acc_sh.at[dst2_v.at[jo]],
                                      ssems.at[ko]).wait()

            @pl.when(j + 1 < NCH)
            def _():
                pltpu.async_copy(h_hbm.at[src_v.at[j + 1]],
                                 rows_v.at[nxt], gsems.at[nxt])

            pltpu.make_async_copy(h_hbm.at[src_v.at[j]],
                                  rows_v.at[cur], gsems.at[cur]).wait()
            pltpu.async_copy(rows_v.at[cur], acc_sh.at[dst2_v.at[j]],
                             ssems.at[cur], add=True)
            return carry

        lax.fori_loop(0, NCH, it, 0)

        for t in range(NB - 1):
            jo = NCH - (NB - 1) + t
            pltpu.make_async_copy(rows_v.at[jo % NB],
                                  acc_sh.at[dst2_v.at[jo]],
                                  ssems.at[jo % NB]).wait()

        plsc.subcore_barrier()
        _writeback(acc_sh, out_hbm, c, s)

    return body(h, srcr, dstr, zeros)


BN = 1000
NBLK = N // BN


def _tc_matmul1(x, W1):
    def body(x_ref, w_ref, g_ref):
        g_ref[...] = jnp.dot(x_ref[...], w_ref[...],
                             preferred_element_type=_F32, precision=_HIGH)

    return pl.pallas_call(
        body,
        grid=(NBLK,),
        in_specs=[
            pl.BlockSpec((BN, D), lambda i: (i, 0)),
            pl.BlockSpec((D, H), lambda i: (0, 0)),
        ],
        out_specs=pl.BlockSpec((BN, H), lambda i: (i, 0)),
        out_shape=jax.ShapeDtypeStruct((N, H), _F32),
    )(x, W1)


def _tc_scale(g1, dp):
    def body(g_ref, dp_ref, h_ref, dinv_ref):
        dinv = lax.rsqrt(dp_ref[:, 0:1] + 1.0)
        dinv_ref[...] = dinv
        h_ref[...] = dinv * g_ref[...]

    return pl.pallas_call(
        body,
        grid=(NBLK,),
        in_specs=[
            pl.BlockSpec((BN, H), lambda i: (i, 0)),
            pl.BlockSpec((BN, DW), lambda i: (i, 0)),
        ],
        out_specs=[
            pl.BlockSpec((BN, H), lambda i: (i, 0)),
            pl.BlockSpec((BN, 1), lambda i: (i, 0)),
        ],
        out_shape=[
            jax.ShapeDtypeStruct((N, H), _F32),
            jax.ShapeDtypeStruct((N, 1), _F32),
        ],
    )(g1, dp)


def _onehot(batch_blk):
    return (batch_blk == lax.broadcasted_iota(jnp.int32, (BN, G), 1)
            ).astype(_F32)


def _tc_stats(agg, hp, dinv, b, batch2):
    def body(a_ref, hp_ref, dinv_ref, b_ref, bat_ref, h_ref, s1_ref,
             sq_ref, cnt_ref):
        i = pl.program_id(0)
        hblk = jnp.maximum(
            dinv_ref[...] * (a_ref[...] + hp_ref[...]) + b_ref[...], 0.0)
        h_ref[...] = hblk
        M = _onehot(bat_ref[...])
        s1 = lax.dot_general(M, hblk, (((0,), (0,)), ((), ())),
                             precision=_HIGH, preferred_element_type=_F32)
        sq = lax.dot_general(M, hblk * hblk, (((0,), (0,)), ((), ())),
                             precision=_HIGH, preferred_element_type=_F32)
        cnt = lax.dot_general(M, jnp.ones((BN, 1), _F32),
                              (((0,), (0,)), ((), ())),
                              precision=_HIGH, preferred_element_type=_F32)

        @pl.when(i == 0)
        def _():
            s1_ref[...] = s1
            sq_ref[...] = sq
            cnt_ref[...] = cnt

        @pl.when(i > 0)
        def _():
            s1_ref[...] += s1
            sq_ref[...] += sq
            cnt_ref[...] += cnt

    return pl.pallas_call(
        body,
        grid=(NBLK,),
        in_specs=[
            pl.BlockSpec((BN, H), lambda i: (i, 0)),
            pl.BlockSpec((BN, H), lambda i: (i, 0)),
            pl.BlockSpec((BN, 1), lambda i: (i, 0)),
            pl.BlockSpec((1, H), lambda i: (0, 0)),
            pl.BlockSpec((BN, 1), lambda i: (i, 0)),
        ],
        out_specs=[
            pl.BlockSpec((BN, H), lambda i: (i, 0)),
            pl.BlockSpec((G, H), lambda i: (0, 0)),
            pl.BlockSpec((G, H), lambda i: (0, 0)),
            pl.BlockSpec((G, 1), lambda i: (0, 0)),
        ],
        out_shape=[
            jax.ShapeDtypeStruct((N, H), _F32),
            jax.ShapeDtypeStruct((G, H), _F32),
            jax.ShapeDtypeStruct((G, H), _F32),
            jax.ShapeDtypeStruct((G, 1), _F32),
        ],
    )(agg, hp, dinv, b, batch2)


def _tc_norm(h, batch2, s1, sq, cnt, gna, gnw, gnb, dinv, Wn):
    def body(h_ref, bat_ref, s1_ref, sq_ref, cnt_ref, a_ref, w_ref, b_ref,
             dinv_ref, wn_ref, out_ref):
        cntc = jnp.maximum(cnt_ref[...], 1.0)
        mean = s1_ref[...] / cntc
        a = a_ref[...]
        var = sq_ref[...] / cntc - (2.0 * a - a * a) * mean * mean
        rstd = lax.rsqrt(var + 1e-5)
        M = _onehot(bat_ref[...])
        meanb = jnp.dot(M, a * mean, preferred_element_type=_F32,
                        precision=_HIGH)
        rstdb = jnp.dot(M, rstd, preferred_element_type=_F32,
                        precision=_HIGH)
        hn = (h_ref[...] - meanb) * rstdb * w_ref[...] + b_ref[...]
        out_ref[...] = dinv_ref[...] * jnp.dot(hn, wn_ref[...],
                                               preferred_element_type=_F32,
                                               precision=_HIGH)

    return pl.pallas_call(
        body,
        grid=(NBLK,),
        in_specs=[
            pl.BlockSpec((BN, H), lambda i: (i, 0)),
            pl.BlockSpec((BN, 1), lambda i: (i, 0)),
            pl.BlockSpec((G, H), lambda i: (0, 0)),
            pl.BlockSpec((G, H), lambda i: (0, 0)),
            pl.BlockSpec((G, 1), lambda i: (0, 0)),
            pl.BlockSpec((1, H), lambda i: (0, 0)),
            pl.BlockSpec((1, H), lambda i: (0, 0)),
            pl.BlockSpec((1, H), lambda i: (0, 0)),
            pl.BlockSpec((BN, 1), lambda i: (i, 0)),
            pl.BlockSpec((H, H), lambda i: (0, 0)),
        ],
        out_specs=pl.BlockSpec((BN, H), lambda i: (i, 0)),
        out_shape=jax.ShapeDtypeStruct((N, H), _F32),
    )(h, batch2, s1, sq, cnt, gna, gnw, gnb, dinv, Wn)


def _tc_final(agg, hp, dinv, b, batch1, Wl, bl):
    def body(a_ref, hp_ref, dinv_ref, b_ref, bat_ref, wl_ref, bl_ref,
             out_ref, h3_ref, bm_ref, pool_ref):
        h3 = dinv_ref[...] * (a_ref[...] + hp_ref[...]) + b_ref[...]
        h3_ref[...] = h3
        bm_ref[...] = jnp.max(h3.reshape(N // 8, 8, H), axis=1)
        pool_ref[...] = jnp.full((G, H), -1e30, _F32)

        def upd(g, row):
            pool_ref[pl.ds(g, 1), :] = jnp.maximum(pool_ref[pl.ds(g, 1), :],
                                                   row)

        def it(j, carry):
            g0 = bat_ref[8 * j]
            g1 = bat_ref[8 * j + 7]

            @pl.when(g0 == g1)
            def _():
                upd(g0, bm_ref[pl.ds(j, 1), :])

            @pl.when(g0 != g1)
            def _():
                for r in range(8):
                    upd(bat_ref[8 * j + r], h3_ref[pl.ds(8 * j + r, 1), :])

            return carry

        lax.fori_loop(0, N // 8, it, 0)
        out_ref[...] = jnp.dot(pool_ref[...], wl_ref[...],
                               preferred_element_type=_F32,
                               precision=_HIGH) + bl_ref[...]

    return pl.pallas_call(
        body,
        in_specs=[
            pl.BlockSpec(memory_space=pltpu.VMEM),
            pl.BlockSpec(memory_space=pltpu.VMEM),
            pl.BlockSpec(memory_space=pltpu.VMEM),
            pl.BlockSpec(memory_space=pltpu.VMEM),
            pl.BlockSpec(memory_space=pltpu.SMEM),
            pl.BlockSpec(memory_space=pltpu.VMEM),
            pl.BlockSpec(memory_space=pltpu.VMEM),
        ],
        out_shape=jax.ShapeDtypeStruct((G, C), _F32),
        scratch_shapes=[
            pltpu.VMEM((N, H), _F32),
            pltpu.VMEM((N // 8, H), _F32),
            pltpu.VMEM((G, H), _F32),
        ],
    )(agg, hp, dinv, b, batch1, Wl, bl)


def kernel(x, edge_index, batch, W1, b1, gn1_w, gn1_b, gn1_a,
           W2, b2, gn2_w, gn2_b, gn2_a, W3, b3, Wl, bl):
    src = edge_index[0].reshape(NS, NCH, K)
    dst = edge_index[1].reshape(NS, NCH, K)
    zeros = jnp.zeros((ACCR, H), _F32)
    ones = jnp.ones((K, DW), _F32)
    batch2 = batch.reshape(N, 1)

    g1 = _tc_matmul1(x, W1)
    dp = _sc_deg(dst, zeros, ones)
    h1p, dinv = _tc_scale(g1, dp)

    a1 = _sc_agg(h1p, src, dst, zeros)
    h1, s1, sq1, cnt = _tc_stats(a1, h1p, dinv, b1.reshape(1, H), batch2)
    h2p = _tc_norm(h1, batch2, s1, sq1, cnt, gn1_a.reshape(1, H),
                   gn1_w.reshape(1, H), gn1_b.reshape(1, H), dinv, W2)

    a2 = _sc_agg(h2p, src, dst, zeros)
    h2, s1b, sq2, cntb = _tc_stats(a2, h2p, dinv, b2.reshape(1, H), batch2)
    h3p = _tc_norm(h2, batch2, s1b, sq2, cntb, gn2_a.reshape(1, H),
                   gn2_w.reshape(1, H), gn2_b.reshape(1, H), dinv, W3)

    a3 = _sc_agg(h3p, src, dst, zeros)
    return _tc_final(a3, h3p, dinv, b3.reshape(1, H), batch,
                     Wl, bl.reshape(1, C))


# speedup vs baseline: 1.0486x; 1.0486x over previous
import functools

import jax
import jax.numpy as jnp
from jax import lax
from jax.experimental import pallas as pl
from jax.experimental.pallas import tpu as pltpu
from jax.experimental.pallas import tpu_sc as plsc

N = 10000
E = 320000
D = 128
H = 128
C = 16
G = 64

NC = 2
NS = 16
NPC = N // NC
ACCR = NPC + NS
EPT = E // NS
K = 80
NCH = EPT // K
DW = 64
NB = 2

WB = 312
WBT = NPC - 15 * WB
ZR = 312
ZRT = ACCR - 15 * ZR

_MESH = dict(core_axis_name="c", subcore_axis_name="s")
_F32 = jnp.float32
_HIGH = lax.Precision.HIGHEST


def _remap(dst_v, dst2_v, c, s):
    base = c * NPC
    dump = NPC + s

    def rm(j, carry):
        for l in range(K // 16):
            v = dst_v[j, pl.ds(l * 16, 16)]
            inr = (v >= base) & (v < base + NPC)
            dst2_v[j, pl.ds(l * 16, 16)] = jnp.where(inr, v - base, dump)
        return carry

    lax.fori_loop(0, NCH, rm, 0)


def _zero_init(z_hbm, acc_sh, s):
    zoff = pl.multiple_of(s * ZR, 8)

    @pl.when(s < 15)
    def _():
        pltpu.sync_copy(z_hbm.at[pl.ds(zoff, ZR)], acc_sh.at[pl.ds(zoff, ZR)])

    @pl.when(s == 15)
    def _():
        pltpu.sync_copy(z_hbm.at[pl.ds(15 * ZR, ZRT)],
                        acc_sh.at[pl.ds(15 * ZR, ZRT)])


def _writeback(acc_sh, out_hbm, c, s):
    woff = pl.multiple_of(s * WB, 8)
    base = pl.multiple_of(c * NPC, 8)

    @pl.when(s < 15)
    def _():
        pltpu.sync_copy(acc_sh.at[pl.ds(woff, WB)],
                        out_hbm.at[pl.ds(base + woff, WB)])

    @pl.when(s == 15)
    def _():
        pltpu.sync_copy(acc_sh.at[pl.ds(15 * WB, WBT)],
                        out_hbm.at[pl.ds(base + 15 * WB, WBT)])


def _sc_deg(dstr, zeros16, ones16):
    @functools.partial(
        pl.kernel,
        out_type=jax.ShapeDtypeStruct((N, DW), _F32),
        mesh=plsc.VectorSubcoreMesh(**_MESH),
        scratch_types=[
            pltpu.VMEM((NCH, K), jnp.int32),
            pltpu.VMEM((NCH, K), jnp.int32),
            pltpu.VMEM((K, DW), _F32),
            pltpu.VMEM_SHARED((ACCR, DW), _F32),
            pltpu.SemaphoreType.DMA,
        ],
    )
    def body(dst_hbm, z_hbm, ones_hbm, out_hbm, dst_v, dst2_v, ones_v,
             acc_sh, sem):
        c = lax.axis_index("c")
        s = lax.axis_index("s")
        pltpu.sync_copy(dst_hbm.at[s], dst_v)
        pltpu.sync_copy(ones_hbm, ones_v)
        _remap(dst_v, dst2_v, c, s)
        _zero_init(z_hbm, acc_sh, s)
        plsc.subcore_barrier()

        def it(j, carry):
            pltpu.sync_copy(ones_v, acc_sh.at[dst2_v.at[j]], add=True)
            return carry

        lax.fori_loop(0, NCH, it, 0)
        plsc.subcore_barrier()
        _writeback(acc_sh, out_hbm, c, s)

    return body(dstr, zeros16, ones16)


def _sc_agg(h, srcr, dstr, zeros):
    @functools.partial(
        pl.kernel,
        out_type=jax.ShapeDtypeStruct((N, H), _F32),
        mesh=plsc.VectorSubcoreMesh(**_MESH),
        scratch_types=[
            pltpu.VMEM((NCH, K), jnp.int32),
            pltpu.VMEM((NCH, K), jnp.int32),
            pltpu.VMEM((NB, K, H), _F32),
            pltpu.VMEM_SHARED((ACCR, H), _F32),
            pltpu.SemaphoreType.DMA((NB,)),
            pltpu.SemaphoreType.DMA((NB,)),
        ],
    )
    def body(h_hbm, src_hbm, dst_hbm, z_hbm, out_hbm,
             src_v, dst2_v, rows_v, acc_sh, gsems, ssems):
        c = lax.axis_index("c")
        s = lax.axis_index("s")
        pltpu.sync_copy(dst_hbm.at[s], dst2_v)
        _remap(dst2_v, dst2_v, c, s)
        pltpu.sync_copy(src_hbm.at[s], src_v)
        _zero_init(z_hbm, acc_sh, s)

        pltpu.async_copy(h_hbm.at[src_v.at[0]], rows_v.at[0], gsems.at[0])
        plsc.subcore_barrier()

        def it(j, carry):
            cur = lax.rem(j, NB)
            nxt = lax.rem(j + 1, NB)

            @pl.when(j >= NB - 1)
            def _():
                jo = j - (NB - 1)
                ko = lax.rem(jo, NB)
                pltpu.make_async_copy(rows_v.at[ko],
                                      acc_sh.at[dst2_v.at[jo]],
                                      ssems.at[ko]).wait()

            @pl.when(j + 1 < NCH)
            def _():
                pltpu.async_copy(h_hbm.at[src_v.at[j + 1]],
                                 rows_v.at[nxt], gsems.at[nxt])

            pltpu.make_async_copy(h_hbm.at[src_v.at[j]],
                                  rows_v.at[cur], gsems.at[cur]).wait()
            pltpu.async_copy(rows_v.at[cur], acc_sh.at[dst2_v.at[j]],
                             ssems.at[cur], add=True)
            return carry

        lax.fori_loop(0, NCH, it, 0)

        for t in range(NB - 1):
            jo = NCH - (NB - 1) + t
            pltpu.make_async_copy(rows_v.at[jo % NB],
                                  acc_sh.at[dst2_v.at[jo]],
                                  ssems.at[jo % NB]).wait()

        plsc.subcore_barrier()
        _writeback(acc_sh, out_hbm, c, s)

    return body(h, srcr, dstr, zeros)


BN = 1000
NBLK = N // BN


def _tc_matmul1(x, W1):
    def body(x_ref, w_ref, g_ref):
        g_ref[...] = jnp.dot(x_ref[...], w_ref[...],
                             preferred_element_type=_F32, precision=_HIGH)

    return pl.pallas_call(
        body,
        grid=(NBLK,),
        in_specs=[
            pl.BlockSpec((BN, D), lambda i: (i, 0)),
            pl.BlockSpec((D, H), lambda i: (0, 0)),
        ],
        out_specs=pl.BlockSpec((BN, H), lambda i: (i, 0)),
        out_shape=jax.ShapeDtypeStruct((N, H), _F32),
    )(x, W1)


def _tc_scale(g1, dp):
    def body(g_ref, dp_ref, h_ref, dinv_ref):
        dinv = lax.rsqrt(dp_ref[:, 0:1] + 1.0)
        dinv_ref[...] = dinv
        h_ref[...] = dinv * g_ref[...]

    return pl.pallas_call(
        body,
        grid=(NBLK,),
        in_specs=[
            pl.BlockSpec((BN, H), lambda i: (i, 0)),
            pl.BlockSpec((BN, DW), lambda i: (i, 0)),
        ],
        out_specs=[
            pl.BlockSpec((BN, H), lambda i: (i, 0)),
            pl.BlockSpec((BN, 1), lambda i: (i, 0)),
        ],
        out_shape=[
            jax.ShapeDtypeStruct((N, H), _F32),
            jax.ShapeDtypeStruct((N, 1), _F32),
        ],
    )(g1, dp)


def _onehot(batch_blk):
    return (batch_blk == lax.broadcasted_iota(jnp.int32, (BN, G), 1)
            ).astype(_F32)


def _tc_stats(agg, hp, dinv, b, batch2):
    def body(a_ref, hp_ref, dinv_ref, b_ref, bat_ref, h_ref, s1_ref,
             sq_ref, cnt_ref):
        i = pl.program_id(0)
        hblk = jnp.maximum(
            dinv_ref[...] * (a_ref[...] + hp_ref[...]) + b_ref[...], 0.0)
        h_ref[...] = hblk
        M = _onehot(bat_ref[...])
        s1 = lax.dot_general(M, hblk, (((0,), (0,)), ((), ())),
                             precision=_HIGH, preferred_element_type=_F32)
        sq = lax.dot_general(M, hblk * hblk, (((0,), (0,)), ((), ())),
                             precision=_HIGH, preferred_element_type=_F32)
        cnt = lax.dot_general(M, jnp.ones((BN, 1), _F32),
                              (((0,), (0,)), ((), ())),
                              precision=_HIGH, preferred_element_type=_F32)

        @pl.when(i == 0)
        def _():
            s1_ref[...] = s1
            sq_ref[...] = sq
            cnt_ref[...] = cnt

        @pl.when(i > 0)
        def _():
            s1_ref[...] += s1
            sq_ref[...] += sq
            cnt_ref[...] += cnt

    return pl.pallas_call(
        body,
        grid=(NBLK,),
        in_specs=[
            pl.BlockSpec((BN, H), lambda i: (i, 0)),
            pl.BlockSpec((BN, H), lambda i: (i, 0)),
            pl.BlockSpec((BN, 1), lambda i: (i, 0)),
            pl.BlockSpec((1, H), lambda i: (0, 0)),
            pl.BlockSpec((BN, 1), lambda i: (i, 0)),
        ],
        out_specs=[
            pl.BlockSpec((BN, H), lambda i: (i, 0)),
            pl.BlockSpec((G, H), lambda i: (0, 0)),
            pl.BlockSpec((G, H), lambda i: (0, 0)),
            pl.BlockSpec((G, 1), lambda i: (0, 0)),
        ],
        out_shape=[
            jax.ShapeDtypeStruct((N, H), _F32),
            jax.ShapeDtypeStruct((G, H), _F32),
            jax.ShapeDtypeStruct((G, H), _F32),
            jax.ShapeDtypeStruct((G, 1), _F32),
        ],
    )(agg, hp, dinv, b, batch2)


def _tc_norm(h, batch2, s1, sq, cnt, gna, gnw, gnb, dinv, Wn):
    def body(h_ref, bat_ref, s1_ref, sq_ref, cnt_ref, a_ref, w_ref, b_ref,
             dinv_ref, wn_ref, out_ref):
        cntc = jnp.maximum(cnt_ref[...], 1.0)
        mean = s1_ref[...] / cntc
        a = a_ref[...]
        var = sq_ref[...] / cntc - (2.0 * a - a * a) * mean * mean
        rstd = lax.rsqrt(var + 1e-5)
        M = _onehot(bat_ref[...])
        meanb = jnp.dot(M, a * mean, preferred_element_type=_F32,
                        precision=_HIGH)
        rstdb = jnp.dot(M, rstd, preferred_element_type=_F32,
                        precision=_HIGH)
        hn = (h_ref[...] - meanb) * rstdb * w_ref[...] + b_ref[...]
        out_ref[...] = dinv_ref[...] * jnp.dot(hn, wn_ref[...],
                                               preferred_element_type=_F32,
                                               precision=_HIGH)

    return pl.pallas_call(
        body,
        grid=(NBLK,),
        in_specs=[
            pl.BlockSpec((BN, H), lambda i: (i, 0)),
            pl.BlockSpec((BN, 1), lambda i: (i, 0)),
            pl.BlockSpec((G, H), lambda i: (0, 0)),
            pl.BlockSpec((G, H), lambda i: (0, 0)),
            pl.BlockSpec((G, 1), lambda i: (0, 0)),
            pl.BlockSpec((1, H), lambda i: (0, 0)),
            pl.BlockSpec((1, H), lambda i: (0, 0)),
            pl.BlockSpec((1, H), lambda i: (0, 0)),
            pl.BlockSpec((BN, 1), lambda i: (i, 0)),
            pl.BlockSpec((H, H), lambda i: (0, 0)),
        ],
        out_specs=pl.BlockSpec((BN, H), lambda i: (i, 0)),
        out_shape=jax.ShapeDtypeStruct((N, H), _F32),
    )(h, batch2, s1, sq, cnt, gna, gnw, gnb, dinv, Wn)


def _tc_final(agg, hp, dinv, b, batch1, Wl, bl):
    def body(a_ref, hp_ref, dinv_ref, b_ref, bat_ref, wl_ref, bl_ref,
             out_ref, h3_ref, bm_ref, pool_ref):
        h3 = dinv_ref[...] * (a_ref[...] + hp_ref[...]) + b_ref[...]
        h3_ref[...] = h3
        bm_ref[...] = jnp.max(h3.reshape(N // 8, 8, H), axis=1)
        pool_ref[...] = jnp.full((G, H), -1e30, _F32)

        def upd(g, row):
            pool_ref[pl.ds(g, 1), :] = jnp.maximum(pool_ref[pl.ds(g, 1), :],
                                                   row)

        def it(j, carry):
            g0 = bat_ref[8 * j]
            g1 = bat_ref[8 * j + 7]

            @pl.when(g0 == g1)
            def _():
                upd(g0, bm_ref[pl.ds(j, 1), :])

            @pl.when(g0 != g1)
            def _():
                for r in range(8):
                    upd(bat_ref[8 * j + r], h3_ref[pl.ds(8 * j + r, 1), :])

            return carry

        lax.fori_loop(0, N // 8, it, 0)
        out_ref[...] = jnp.dot(pool_ref[...], wl_ref[...],
                               preferred_element_type=_F32,
                               precision=_HIGH) + bl_ref[...]

    return pl.pallas_call(
        body,
        in_specs=[
            pl.BlockSpec(memory_space=pltpu.VMEM),
            pl.BlockSpec(memory_space=pltpu.VMEM),
            pl.BlockSpec(memory_space=pltpu.VMEM),
            pl.BlockSpec(memory_space=pltpu.VMEM),
            pl.BlockSpec(memory_space=pltpu.SMEM),
            pl.BlockSpec(memory_space=pltpu.VMEM),
            pl.BlockSpec(memory_space=pltpu.VMEM),
        ],
        out_shape=jax.ShapeDtypeStruct((G, C), _F32),
        scratch_shapes=[
            pltpu.VMEM((N, H), _F32),
            pltpu.VMEM((N // 8, H), _F32),
            pltpu.VMEM((G, H), _F32),
        ],
    )(agg, hp, dinv, b, batch1, Wl, bl)


def kernel(x, edge_index, batch, W1, b1, gn1_w, gn1_b, gn1_a,
           W2, b2, gn2_w, gn2_b, gn2_a, W3, b3, Wl, bl):
    src = edge_index[0].reshape(NS, NCH, K)
    dst = edge_index[1].reshape(NS, NCH, K)
    zeros = jnp.zeros((ACCR, H), _F32)
    zerosd = jnp.zeros((ACCR, DW), _F32)
    ones = jnp.ones((K, DW), _F32)
    batch2 = batch.reshape(N, 1)

    g1 = _tc_matmul1(x, W1)
    dp = _sc_deg(dst, zerosd, ones)
    h1p, dinv = _tc_scale(g1, dp)

    a1 = _sc_agg(h1p, src, dst, zeros)
    h1, s1, sq1, cnt = _tc_stats(a1, h1p, dinv, b1.reshape(1, H), batch2)
    h2p = _tc_norm(h1, batch2, s1, sq1, cnt, gn1_a.reshape(1, H),
                   gn1_w.reshape(1, H), gn1_b.reshape(1, H), dinv, W2)

    a2 = _sc_agg(h2p, src, dst, zeros)
    h2, s1b, sq2, cntb = _tc_stats(a2, h2p, dinv, b2.reshape(1, H), batch2)
    h3p = _tc_norm(h2, batch2, s1b, sq2, cntb, gn2_a.reshape(1, H),
                   gn2_w.reshape(1, H), gn2_b.reshape(1, H), dinv, W3)

    a3 = _sc_agg(h3p, src, dst, zeros)
    return _tc_final(a3, h3p, dinv, b3.reshape(1, H), batch,
                     Wl, bl.reshape(1, C))


# speedup vs baseline: 1.0742x; 1.0244x over previous
import functools

import jax
import jax.numpy as jnp
from jax import lax
from jax.experimental import pallas as pl
from jax.experimental.pallas import tpu as pltpu
from jax.experimental.pallas import tpu_sc as plsc

N = 10000
E = 320000
D = 128
H = 128
C = 16
G = 64

NC = 2
NS = 16
NPC = N // NC
ACCR = NPC + NS
EPT = E // NS
K = 80
NCH = EPT // K
DW = 32
NB = 2

WB = 312
WBT = NPC - 15 * WB
ZR = 312
ZRT = ACCR - 15 * ZR

_MESH = dict(core_axis_name="c", subcore_axis_name="s")
_F32 = jnp.float32
_HIGH = lax.Precision.HIGHEST


def _remap(dst_v, dst2_v, c, s):
    base = c * NPC
    dump = NPC + s

    def rm(j, carry):
        for l in range(K // 16):
            v = dst_v[j, pl.ds(l * 16, 16)]
            inr = (v >= base) & (v < base + NPC)
            dst2_v[j, pl.ds(l * 16, 16)] = jnp.where(inr, v - base, dump)
        return carry

    lax.fori_loop(0, NCH, rm, 0)


def _zero_init(z_hbm, acc_sh, s):
    zoff = pl.multiple_of(s * ZR, 8)

    @pl.when(s < 15)
    def _():
        pltpu.sync_copy(z_hbm.at[pl.ds(zoff, ZR)], acc_sh.at[pl.ds(zoff, ZR)])

    @pl.when(s == 15)
    def _():
        pltpu.sync_copy(z_hbm.at[pl.ds(15 * ZR, ZRT)],
                        acc_sh.at[pl.ds(15 * ZR, ZRT)])


def _writeback(acc_sh, out_hbm, c, s):
    woff = pl.multiple_of(s * WB, 8)
    base = pl.multiple_of(c * NPC, 8)

    @pl.when(s < 15)
    def _():
        pltpu.sync_copy(acc_sh.at[pl.ds(woff, WB)],
                        out_hbm.at[pl.ds(base + woff, WB)])

    @pl.when(s == 15)
    def _():
        pltpu.sync_copy(acc_sh.at[pl.ds(15 * WB, WBT)],
                        out_hbm.at[pl.ds(base + 15 * WB, WBT)])


def _sc_deg(dstr, zeros16, ones16):
    @functools.partial(
        pl.kernel,
        out_type=jax.ShapeDtypeStruct((N, DW), _F32),
        mesh=plsc.VectorSubcoreMesh(**_MESH),
        scratch_types=[
            pltpu.VMEM((NCH, K), jnp.int32),
            pltpu.VMEM((NCH, K), jnp.int32),
            pltpu.VMEM((K, DW), _F32),
            pltpu.VMEM_SHARED((ACCR, DW), _F32),
            pltpu.SemaphoreType.DMA,
        ],
    )
    def body(dst_hbm, z_hbm, ones_hbm, out_hbm, dst_v, dst2_v, ones_v,
             acc_sh, sem):
        c = lax.axis_index("c")
        s = lax.axis_index("s")
        pltpu.sync_copy(dst_hbm.at[s], dst_v)
        pltpu.sync_copy(ones_hbm, ones_v)
        _remap(dst_v, dst2_v, c, s)
        _zero_init(z_hbm, acc_sh, s)
        plsc.subcore_barrier()

        def it(j, carry):
            pltpu.sync_copy(ones_v, acc_sh.at[dst2_v.at[j]], add=True)
            return carry

        lax.fori_loop(0, NCH, it, 0)
        plsc.subcore_barrier()
        _writeback(acc_sh, out_hbm, c, s)

    return body(dstr, zeros16, ones16)


def _sc_agg(h, srcr, dstr, zeros):
    @functools.partial(
        pl.kernel,
        out_type=jax.ShapeDtypeStruct((N, H), _F32),
        mesh=plsc.VectorSubcoreMesh(**_MESH),
        scratch_types=[
            pltpu.VMEM((NCH, K), jnp.int32),
            pltpu.VMEM((NCH, K), jnp.int32),
            pltpu.VMEM((NB, K, H), _F32),
            pltpu.VMEM_SHARED((ACCR, H), _F32),
            pltpu.SemaphoreType.DMA((NB,)),
            pltpu.SemaphoreType.DMA((NB,)),
        ],
    )
    def body(h_hbm, src_hbm, dst_hbm, z_hbm, out_hbm,
             src_v, dst2_v, rows_v, acc_sh, gsems, ssems):
        c = lax.axis_index("c")
        s = lax.axis_index("s")
        pltpu.sync_copy(dst_hbm.at[s], dst2_v)
        _remap(dst2_v, dst2_v, c, s)
        pltpu.sync_copy(src_hbm.at[s], src_v)
        _zero_init(z_hbm, acc_sh, s)

        pltpu.async_copy(h_hbm.at[src_v.at[0]], rows_v.at[0], gsems.at[0])
        plsc.subcore_barrier()

        def it(j, carry):
            cur = lax.rem(j, NB)
            nxt = lax.rem(j + 1, NB)

            @pl.when(j >= NB - 1)
            def _():
                jo = j - (NB - 1)
                ko = lax.rem(jo, NB)
                pltpu.make_async_copy(rows_v.at[ko],
                                      acc_sh.at[dst2_v.at[jo]],
                                      ssems.at[ko]).wait()

            @pl.when(j + 1 < NCH)
            def _():
                pltpu.async_copy(h_hbm.at[src_v.at[j + 1]],
                                 rows_v.at[nxt], gsems.at[nxt])

            pltpu.make_async_copy(h_hbm.at[src_v.at[j]],
                                  rows_v.at[cur], gsems.at[cur]).wait()
            pltpu.async_copy(rows_v.at[cur], acc_sh.at[dst2_v.at[j]],
                             ssems.at[cur], add=True)
            return carry

        lax.fori_loop(0, NCH, it, 0)

        for t in range(NB - 1):
            jo = NCH - (NB - 1) + t
            pltpu.make_async_copy(rows_v.at[jo % NB],
                                  acc_sh.at[dst2_v.at[jo]],
                                  ssems.at[jo % NB]).wait()

        plsc.subcore_barrier()
        _writeback(acc_sh, out_hbm, c, s)

    return body(h, srcr, dstr, zeros)


BN = 1000
NBLK = N // BN


def _tc_matmul1(x, W1):
    def body(x_ref, w_ref, g_ref):
        g_ref[...] = jnp.dot(x_ref[...], w_ref[...],
                             preferred_element_type=_F32, precision=_HIGH)

    return pl.pallas_call(
        body,
        grid=(NBLK,),
        in_specs=[
            pl.BlockSpec((BN, D), lambda i: (i, 0)),
            pl.BlockSpec((D, H), lambda i: (0, 0)),
        ],
        out_specs=pl.BlockSpec((BN, H), lambda i: (i, 0)),
        out_shape=jax.ShapeDtypeStruct((N, H), _F32),
    )(x, W1)


def _tc_scale(g1, dp):
    def body(g_ref, dp_ref, h_ref, dinv_ref):
        dinv = lax.rsqrt(dp_ref[:, 0:1] + 1.0)
        dinv_ref[...] = dinv
        h_ref[...] = dinv * g_ref[...]

    return pl.pallas_call(
        body,
        grid=(NBLK,),
        in_specs=[
            pl.BlockSpec((BN, H), lambda i: (i, 0)),
            pl.BlockSpec((BN, DW), lambda i: (i, 0)),
        ],
        out_specs=[
            pl.BlockSpec((BN, H), lambda i: (i, 0)),
            pl.BlockSpec((BN, 1), lambda i: (i, 0)),
        ],
        out_shape=[
            jax.ShapeDtypeStruct((N, H), _F32),
            jax.ShapeDtypeStruct((N, 1), _F32),
        ],
    )(g1, dp)


def _onehot(batch_blk):
    return (batch_blk == lax.broadcasted_iota(jnp.int32, (BN, G), 1)
            ).astype(_F32)


def _tc_stats(agg, hp, dinv, b, batch2):
    def body(a_ref, hp_ref, dinv_ref, b_ref, bat_ref, h_ref, s1_ref,
             sq_ref, cnt_ref):
        i = pl.program_id(0)
        hblk = jnp.maximum(
            dinv_ref[...] * (a_ref[...] + hp_ref[...]) + b_ref[...], 0.0)
        h_ref[...] = hblk
        M = _onehot(bat_ref[...])
        s1 = lax.dot_general(M, hblk, (((0,), (0,)), ((), ())),
                             precision=_HIGH, preferred_element_type=_F32)
        sq = lax.dot_general(M, hblk * hblk, (((0,), (0,)), ((), ())),
                             precision=_HIGH, preferred_element_type=_F32)
        cnt = lax.dot_general(M, jnp.ones((BN, 1), _F32),
                              (((0,), (0,)), ((), ())),
                              precision=_HIGH, preferred_element_type=_F32)

        @pl.when(i == 0)
        def _():
            s1_ref[...] = s1
            sq_ref[...] = sq
            cnt_ref[...] = cnt

        @pl.when(i > 0)
        def _():
            s1_ref[...] += s1
            sq_ref[...] += sq
            cnt_ref[...] += cnt

    return pl.pallas_call(
        body,
        grid=(NBLK,),
        in_specs=[
            pl.BlockSpec((BN, H), lambda i: (i, 0)),
            pl.BlockSpec((BN, H), lambda i: (i, 0)),
            pl.BlockSpec((BN, 1), lambda i: (i, 0)),
            pl.BlockSpec((1, H), lambda i: (0, 0)),
            pl.BlockSpec((BN, 1), lambda i: (i, 0)),
        ],
        out_specs=[
            pl.BlockSpec((BN, H), lambda i: (i, 0)),
            pl.BlockSpec((G, H), lambda i: (0, 0)),
            pl.BlockSpec((G, H), lambda i: (0, 0)),
            pl.BlockSpec((G, 1), lambda i: (0, 0)),
        ],
        out_shape=[
            jax.ShapeDtypeStruct((N, H), _F32),
            jax.ShapeDtypeStruct((G, H), _F32),
            jax.ShapeDtypeStruct((G, H), _F32),
            jax.ShapeDtypeStruct((G, 1), _F32),
        ],
    )(agg, hp, dinv, b, batch2)


def _tc_norm(h, batch2, s1, sq, cnt, gna, gnw, gnb, dinv, Wn):
    def body(h_ref, bat_ref, s1_ref, sq_ref, cnt_ref, a_ref, w_ref, b_ref,
             dinv_ref, wn_ref, out_ref):
        cntc = jnp.maximum(cnt_ref[...], 1.0)
        mean = s1_ref[...] / cntc
        a = a_ref[...]
        var = sq_ref[...] / cntc - (2.0 * a - a * a) * mean * mean
        rstd = lax.rsqrt(var + 1e-5)
        M = _onehot(bat_ref[...])
        meanb = jnp.dot(M, a * mean, preferred_element_type=_F32,
                        precision=_HIGH)
        rstdb = jnp.dot(M, rstd, preferred_element_type=_F32,
                        precision=_HIGH)
        hn = (h_ref[...] - meanb) * rstdb * w_ref[...] + b_ref[...]
        out_ref[...] = dinv_ref[...] * jnp.dot(hn, wn_ref[...],
                                               preferred_element_type=_F32,
                                               precision=_HIGH)

    return pl.pallas_call(
        body,
        grid=(NBLK,),
        in_specs=[
            pl.BlockSpec((BN, H), lambda i: (i, 0)),
            pl.BlockSpec((BN, 1), lambda i: (i, 0)),
            pl.BlockSpec((G, H), lambda i: (0, 0)),
            pl.BlockSpec((G, H), lambda i: (0, 0)),
            pl.BlockSpec((G, 1), lambda i: (0, 0)),
            pl.BlockSpec((1, H), lambda i: (0, 0)),
            pl.BlockSpec((1, H), lambda i: (0, 0)),
            pl.BlockSpec((1, H), lambda i: (0, 0)),
            pl.BlockSpec((BN, 1), lambda i: (i, 0)),
            pl.BlockSpec((H, H), lambda i: (0, 0)),
        ],
        out_specs=pl.BlockSpec((BN, H), lambda i: (i, 0)),
        out_shape=jax.ShapeDtypeStruct((N, H), _F32),
    )(h, batch2, s1, sq, cnt, gna, gnw, gnb, dinv, Wn)


def _tc_final(agg, hp, dinv, b, batch1, Wl, bl):
    def body(a_ref, hp_ref, dinv_ref, b_ref, bat_ref, wl_ref, bl_ref,
             out_ref, h3_ref, bm_ref, pool_ref):
        h3 = dinv_ref[...] * (a_ref[...] + hp_ref[...]) + b_ref[...]
        h3_ref[...] = h3
        bm_ref[...] = jnp.max(h3.reshape(N // 8, 8, H), axis=1)
        pool_ref[...] = jnp.full((G, H), -1e30, _F32)

        def upd(g, row):
            pool_ref[pl.ds(g, 1), :] = jnp.maximum(pool_ref[pl.ds(g, 1), :],
                                                   row)

        def it(j, carry):
            g0 = bat_ref[8 * j]
            g1 = bat_ref[8 * j + 7]

            @pl.when(g0 == g1)
            def _():
                upd(g0, bm_ref[pl.ds(j, 1), :])

            @pl.when(g0 != g1)
            def _():
                for r in range(8):
                    upd(bat_ref[8 * j + r], h3_ref[pl.ds(8 * j + r, 1), :])

            return carry

        lax.fori_loop(0, N // 8, it, 0)
        out_ref[...] = jnp.dot(pool_ref[...], wl_ref[...],
                               preferred_element_type=_F32,
                               precision=_HIGH) + bl_ref[...]

    return pl.pallas_call(
        body,
        in_specs=[
            pl.BlockSpec(memory_space=pltpu.VMEM),
            pl.BlockSpec(memory_space=pltpu.VMEM),
            pl.BlockSpec(memory_space=pltpu.VMEM),
            pl.BlockSpec(memory_space=pltpu.VMEM),
            pl.BlockSpec(memory_space=pltpu.SMEM),
            pl.BlockSpec(memory_space=pltpu.VMEM),
            pl.BlockSpec(memory_space=pltpu.VMEM),
        ],
        out_shape=jax.ShapeDtypeStruct((G, C), _F32),
        scratch_shapes=[
            pltpu.VMEM((N, H), _F32),
            pltpu.VMEM((N // 8, H), _F32),
            pltpu.VMEM((G, H), _F32),
        ],
    )(agg, hp, dinv, b, batch1, Wl, bl)


def kernel(x, edge_index, batch, W1, b1, gn1_w, gn1_b, gn1_a,
           W2, b2, gn2_w, gn2_b, gn2_a, W3, b3, Wl, bl):
    src = edge_index[0].reshape(NS, NCH, K)
    dst = edge_index[1].reshape(NS, NCH, K)
    zeros = jnp.zeros((ACCR, H), _F32)
    zerosd = jnp.zeros((ACCR, DW), _F32)
    ones = jnp.ones((K, DW), _F32)
    batch2 = batch.reshape(N, 1)

    g1 = _tc_matmul1(x, W1)
    dp = _sc_deg(dst, zerosd, ones)
    h1p, dinv = _tc_scale(g1, dp)

    a1 = _sc_agg(h1p, src, dst, zeros)
    h1, s1, sq1, cnt = _tc_stats(a1, h1p, dinv, b1.reshape(1, H), batch2)
    h2p = _tc_norm(h1, batch2, s1, sq1, cnt, gn1_a.reshape(1, H),
                   gn1_w.reshape(1, H), gn1_b.reshape(1, H), dinv, W2)

    a2 = _sc_agg(h2p, src, dst, zeros)
    h2, s1b, sq2, cntb = _tc_stats(a2, h2p, dinv, b2.reshape(1, H), batch2)
    h3p = _tc_norm(h2, batch2, s1b, sq2, cntb, gn2_a.reshape(1, H),
                   gn2_w.reshape(1, H), gn2_b.reshape(1, H), dinv, W3)

    a3 = _sc_agg(h3p, src, dst, zeros)
    return _tc_final(a3, h3p, dinv, b3.reshape(1, H), batch,
                     Wl, bl.reshape(1, C))
